# Initial kernel scaffold; baseline (speedup 1.0000x reference)
import jax
import jax.numpy as jnp
from jax.experimental import pallas as pl


def kernel(inputs, codebook):
    raise NotImplementedError("write your pallas kernel here")


# speedup vs baseline: 1.0599x; 1.0599x over previous
import functools

import jax
import jax.numpy as jnp
from jax import lax
from jax.experimental import pallas as pl
from jax.experimental.pallas import tpu as pltpu
from jax.experimental.pallas import tpu_sc as plsc

N_TOKENS = 16384
EMBED_DIM = 32
NUM_CODES = 8192
COMMITMENT_COST = 0.25

TOKEN_TILE = 256


def _half_min_argmin(dist, base):
    minv = jnp.min(dist, axis=1, keepdims=True)
    kiota = lax.broadcasted_iota(jnp.int32, dist.shape, 1)
    idx = jnp.min(jnp.where(dist == minv, kiota, NUM_CODES), axis=1) + base
    return minv[:, 0], idx


def _dist_argmin_kernel(x_ref, cb_ref, idx_ref, loss_ref):
    i = pl.program_id(0)
    x = x_ref[...]
    c = cb_ref[...]
    s1 = jnp.sum(x * x, axis=1, keepdims=True)
    s2 = jnp.sum(c * c, axis=1)
    m = lax.dot_general(x, c, (((1,), (1,)), ((), ())),
                        preferred_element_type=jnp.float32)
    dist = (s1 + s2[None, :]) - 2.0 * m
    half = NUM_CODES // 2
    vL, iL = _half_min_argmin(dist[:, :half], 0)
    vR, iR = _half_min_argmin(dist[:, half:], half)
    vLb = vL.astype(jnp.bfloat16).astype(jnp.float32)
    pick_l = vLb <= vR
    idx = jnp.where(pick_l, iL, iR)
    minv = jnp.where(pick_l, vL, vR)
    idx_ref[0, 0, :] = idx

    @pl.when(i == 0)
    def _init():
        loss_ref[...] = jnp.zeros((1, 1), jnp.float32)

    loss_ref[...] += jnp.sum(minv).reshape(1, 1)

    @pl.when(i == pl.num_programs(0) - 1)
    def _finalize():
        scale = (1.0 + COMMITMENT_COST) / (N_TOKENS * EMBED_DIM)
        loss_ref[...] = loss_ref[...] * scale


def _dist_argmin(inputs, codebook):
    n_tiles = N_TOKENS // TOKEN_TILE
    idx3, loss = pl.pallas_call(
        _dist_argmin_kernel,
        grid=(n_tiles,),
        in_specs=[
            pl.BlockSpec((TOKEN_TILE, EMBED_DIM), lambda i: (i, 0)),
            pl.BlockSpec((NUM_CODES, EMBED_DIM), lambda i: (0, 0)),
        ],
        out_specs=[
            pl.BlockSpec((1, 1, TOKEN_TILE), lambda i: (i, 0, 0)),
            pl.BlockSpec((1, 1), lambda i: (0, 0)),
        ],
        out_shape=[
            jax.ShapeDtypeStruct((n_tiles, 1, TOKEN_TILE), jnp.int32),
            jax.ShapeDtypeStruct((1, 1), jnp.float32),
        ],
    )(inputs, codebook)
    return idx3.reshape(N_TOKENS), loss[0, 0]


GATHER_WIDTH = 128


def _make_sc_gather():
    info = plsc.get_sparse_core_info()
    nc, ns = info.num_cores, info.num_subcores
    nw = nc * ns
    b_per_w = N_TOKENS // nw
    chunk = 128
    n_chunks = b_per_w // chunk
    mesh = plsc.VectorSubcoreMesh(core_axis_name="c", subcore_axis_name="s")

    @functools.partial(
        pl.kernel,
        mesh=mesh,
        out_type=jax.ShapeDtypeStruct((N_TOKENS, GATHER_WIDTH), jnp.float32),
        scratch_types=[
            pltpu.VMEM((n_chunks, chunk), jnp.int32),
            pltpu.VMEM((chunk, GATHER_WIDTH), jnp.float32),
            pltpu.SemaphoreType.DMA,
        ],
    )
    def gather_kernel(table_hbm, idx_hbm, out_hbm, idx_v, rows_v, sem):
        wid = lax.axis_index("s") * nc + lax.axis_index("c")
        base = wid * b_per_w
        for j in range(n_chunks):
            pltpu.sync_copy(idx_hbm.at[pl.ds(base + j * chunk, chunk)],
                            idx_v.at[j])
            pltpu.async_copy(table_hbm.at[idx_v.at[j]], rows_v, sem).wait()
            pltpu.sync_copy(rows_v,
                            out_hbm.at[pl.ds(base + j * chunk, chunk)])

    return gather_kernel


def kernel(inputs, codebook):
    encoding_indices, loss = _dist_argmin(inputs, codebook)
    table = jnp.pad(codebook, ((0, 0), (0, GATHER_WIDTH - EMBED_DIM)))
    quantized = _make_sc_gather()(table, encoding_indices)[:, :EMBED_DIM]
    return (quantized, loss, encoding_indices)


# speedup vs baseline: 1.2765x; 1.2044x over previous
import functools

import jax
import jax.numpy as jnp
from jax import lax
from jax.experimental import pallas as pl
from jax.experimental.pallas import tpu as pltpu
from jax.experimental.pallas import tpu_sc as plsc

N_TOKENS = 16384
EMBED_DIM = 32
NUM_CODES = 8192
COMMITMENT_COST = 0.25

TOKEN_TILE = 256


def _row_sumsq(a):
    sq = a * a
    t = ((sq[:, 0:8] + sq[:, 8:16]) + sq[:, 16:24]) + sq[:, 24:32]
    u = t[:, 0:4] + t[:, 4:8]
    w = u[:, 0:2] + u[:, 2:4]
    return w[:, 0:1] + w[:, 1:2]


def _half_min_argmin(dist, iota_f):
    minv = jnp.min(dist, axis=1, keepdims=True)
    idx_f = jnp.min(jnp.where(dist == minv, iota_f, 1e9), axis=1)
    return minv[:, 0], idx_f.astype(jnp.int32)


def _dist_argmin_kernel(x_ref, cb_ref, idx_ref, loss_ref, s2_ref, iota_ref):
    i = pl.program_id(0)

    @pl.when(i == 0)
    def _precompute():
        c = cb_ref[...]
        s2_ref[...] = _row_sumsq(c).reshape(1, NUM_CODES)
        iota_ref[...] = lax.broadcasted_iota(
            jnp.int32, (1, NUM_CODES), 1).astype(jnp.float32)

    x = x_ref[...]
    s1 = _row_sumsq(x)
    m2 = lax.dot_general(-2.0 * x, cb_ref[...], (((1,), (1,)), ((), ())),
                         preferred_element_type=jnp.float32)
    dist = (s1 + s2_ref[...]) + m2
    half = NUM_CODES // 2
    iota_f = iota_ref[...]
    vL, iL = _half_min_argmin(dist[:, :half], iota_f[:, :half])
    vR, iR = _half_min_argmin(dist[:, half:], iota_f[:, half:])
    vLb = vL.astype(jnp.bfloat16).astype(jnp.float32)
    pick_l = vLb <= vR
    idx = jnp.where(pick_l, iL, iR)
    minv = jnp.where(pick_l, vL, vR)
    idx_ref[0, 0, :] = idx

    @pl.when(i == 0)
    def _init():
        loss_ref[...] = jnp.zeros((1, 1), jnp.float32)

    loss_ref[...] += jnp.sum(minv).reshape(1, 1)

    @pl.when(i == pl.num_programs(0) - 1)
    def _finalize():
        scale = (1.0 + COMMITMENT_COST) / (N_TOKENS * EMBED_DIM)
        loss_ref[...] = loss_ref[...] * scale


def _dist_argmin(inputs, codebook):
    n_tiles = N_TOKENS // TOKEN_TILE
    idx3, loss = pl.pallas_call(
        _dist_argmin_kernel,
        grid=(n_tiles,),
        in_specs=[
            pl.BlockSpec((TOKEN_TILE, EMBED_DIM), lambda i: (i, 0)),
            pl.BlockSpec((NUM_CODES, EMBED_DIM), lambda i: (0, 0)),
        ],
        out_specs=[
            pl.BlockSpec((1, 1, TOKEN_TILE), lambda i: (i, 0, 0)),
            pl.BlockSpec((1, 1), lambda i: (0, 0)),
        ],
        out_shape=[
            jax.ShapeDtypeStruct((n_tiles, 1, TOKEN_TILE), jnp.int32),
            jax.ShapeDtypeStruct((1, 1), jnp.float32),
        ],
        scratch_shapes=[
            pltpu.VMEM((1, NUM_CODES), jnp.float32),
            pltpu.VMEM((1, NUM_CODES), jnp.float32),
        ],
    )(inputs, codebook)
    return idx3.reshape(N_TOKENS), loss[0, 0]


GATHER_WIDTH = 128


def _make_sc_gather():
    info = plsc.get_sparse_core_info()
    nc, ns = info.num_cores, info.num_subcores
    nw = nc * ns
    b_per_w = N_TOKENS // nw
    chunk = 128
    n_chunks = b_per_w // chunk
    mesh = plsc.VectorSubcoreMesh(core_axis_name="c", subcore_axis_name="s")

    @functools.partial(
        pl.kernel,
        mesh=mesh,
        out_type=jax.ShapeDtypeStruct((N_TOKENS, GATHER_WIDTH), jnp.float32),
        scratch_types=[
            pltpu.VMEM((n_chunks, chunk), jnp.int32),
            pltpu.VMEM((chunk, GATHER_WIDTH), jnp.float32),
            pltpu.SemaphoreType.DMA,
        ],
    )
    def gather_kernel(table_hbm, idx_hbm, out_hbm, idx_v, rows_v, sem):
        wid = lax.axis_index("s") * nc + lax.axis_index("c")
        base = wid * b_per_w
        for j in range(n_chunks):
            pltpu.sync_copy(idx_hbm.at[pl.ds(base + j * chunk, chunk)],
                            idx_v.at[j])
            pltpu.async_copy(table_hbm.at[idx_v.at[j]], rows_v, sem).wait()
            pltpu.sync_copy(rows_v,
                            out_hbm.at[pl.ds(base + j * chunk, chunk)])

    return gather_kernel


def kernel(inputs, codebook):
    encoding_indices, loss = _dist_argmin(inputs, codebook)
    table = jnp.pad(codebook, ((0, 0), (0, GATHER_WIDTH - EMBED_DIM)))
    quantized = _make_sc_gather()(table, encoding_indices)[:, :EMBED_DIM]
    return (quantized, loss, encoding_indices)


# speedup vs baseline: 1.3243x; 1.0375x over previous
import functools

import jax
import jax.numpy as jnp
from jax import lax
from jax.experimental import pallas as pl
from jax.experimental.pallas import tpu as pltpu
from jax.experimental.pallas import tpu_sc as plsc

N_TOKENS = 16384
EMBED_DIM = 32
NUM_CODES = 8192
COMMITMENT_COST = 0.25

TOKEN_TILE = 512


def _row_sumsq(a):
    sq = a * a
    t = ((sq[:, 0:8] + sq[:, 8:16]) + sq[:, 16:24]) + sq[:, 24:32]
    u = t[:, 0:4] + t[:, 4:8]
    w = u[:, 0:2] + u[:, 2:4]
    return w[:, 0:1] + w[:, 1:2]


def _half_min_argmin(dist, iota_f):
    minv = jnp.min(dist, axis=1, keepdims=True)
    idx_f = jnp.min(jnp.where(dist == minv, iota_f, 1e9), axis=1)
    return minv[:, 0], idx_f.astype(jnp.int32)


def _dist_argmin_kernel(x_ref, cb_ref, idx_ref, loss_ref, s2_ref, iota_ref):
    i = pl.program_id(0)

    @pl.when(i == 0)
    def _precompute():
        c = cb_ref[...]
        s2_ref[...] = _row_sumsq(c).reshape(1, NUM_CODES)
        iota_ref[...] = lax.broadcasted_iota(
            jnp.int32, (1, NUM_CODES), 1).astype(jnp.float32)

    x = x_ref[...]
    s1 = _row_sumsq(x)
    m2 = lax.dot_general(-2.0 * x, cb_ref[...], (((1,), (1,)), ((), ())),
                         preferred_element_type=jnp.float32)
    dist = (s1 + s2_ref[...]) + m2
    half = NUM_CODES // 2
    iota_f = iota_ref[...]
    vL, iL = _half_min_argmin(dist[:, :half], iota_f[:, :half])
    vR, iR = _half_min_argmin(dist[:, half:], iota_f[:, half:])
    vLb = vL.astype(jnp.bfloat16).astype(jnp.float32)
    pick_l = vLb <= vR
    idx = jnp.where(pick_l, iL, iR)
    minv = jnp.where(pick_l, vL, vR)
    idx_ref[0, 0, :] = idx

    @pl.when(i == 0)
    def _init():
        loss_ref[...] = jnp.zeros((1, 1), jnp.float32)

    loss_ref[...] += jnp.sum(minv).reshape(1, 1)

    @pl.when(i == pl.num_programs(0) - 1)
    def _finalize():
        scale = (1.0 + COMMITMENT_COST) / (N_TOKENS * EMBED_DIM)
        loss_ref[...] = loss_ref[...] * scale


def _dist_argmin(inputs, codebook):
    n_tiles = N_TOKENS // TOKEN_TILE
    idx3, loss = pl.pallas_call(
        _dist_argmin_kernel,
        grid=(n_tiles,),
        in_specs=[
            pl.BlockSpec((TOKEN_TILE, EMBED_DIM), lambda i: (i, 0)),
            pl.BlockSpec((NUM_CODES, EMBED_DIM), lambda i: (0, 0)),
        ],
        out_specs=[
            pl.BlockSpec((1, 1, TOKEN_TILE), lambda i: (i, 0, 0)),
            pl.BlockSpec((1, 1), lambda i: (0, 0)),
        ],
        out_shape=[
            jax.ShapeDtypeStruct((n_tiles, 1, TOKEN_TILE), jnp.int32),
            jax.ShapeDtypeStruct((1, 1), jnp.float32),
        ],
        scratch_shapes=[
            pltpu.VMEM((1, NUM_CODES), jnp.float32),
            pltpu.VMEM((1, NUM_CODES), jnp.float32),
        ],
    )(inputs, codebook)
    return idx3.reshape(N_TOKENS), loss[0, 0]


GATHER_WIDTH = 128


def _make_sc_gather():
    info = plsc.get_sparse_core_info()
    nc, ns = info.num_cores, info.num_subcores
    nw = nc * ns
    b_per_w = N_TOKENS // nw
    chunk = 128
    n_chunks = b_per_w // chunk
    mesh = plsc.VectorSubcoreMesh(core_axis_name="c", subcore_axis_name="s")

    @functools.partial(
        pl.kernel,
        mesh=mesh,
        out_type=jax.ShapeDtypeStruct((N_TOKENS, GATHER_WIDTH), jnp.float32),
        scratch_types=[
            pltpu.VMEM((n_chunks, chunk), jnp.int32),
            pltpu.VMEM((chunk, GATHER_WIDTH), jnp.float32),
            pltpu.SemaphoreType.DMA,
        ],
    )
    def gather_kernel(table_hbm, idx_hbm, out_hbm, idx_v, rows_v, sem):
        wid = lax.axis_index("s") * nc + lax.axis_index("c")
        base = wid * b_per_w
        for j in range(n_chunks):
            pltpu.sync_copy(idx_hbm.at[pl.ds(base + j * chunk, chunk)],
                            idx_v.at[j])
            pltpu.async_copy(table_hbm.at[idx_v.at[j]], rows_v, sem).wait()
            pltpu.sync_copy(rows_v,
                            out_hbm.at[pl.ds(base + j * chunk, chunk)])

    return gather_kernel


def kernel(inputs, codebook):
    encoding_indices, loss = _dist_argmin(inputs, codebook)
    table = jnp.pad(codebook, ((0, 0), (0, GATHER_WIDTH - EMBED_DIM)))
    quantized = _make_sc_gather()(table, encoding_indices)[:, :EMBED_DIM]
    return (quantized, loss, encoding_indices)


# speedup vs baseline: 1.3613x; 1.0279x over previous
import functools

import jax
import jax.numpy as jnp
from jax import lax
from jax.experimental import pallas as pl
from jax.experimental.pallas import tpu as pltpu
from jax.experimental.pallas import tpu_sc as plsc

N_TOKENS = 16384
EMBED_DIM = 32
NUM_CODES = 8192
COMMITMENT_COST = 0.25

TOKEN_TILE = 1024


def _row_sumsq(a):
    sq = a * a
    t = ((sq[:, 0:8] + sq[:, 8:16]) + sq[:, 16:24]) + sq[:, 24:32]
    u = t[:, 0:4] + t[:, 4:8]
    w = u[:, 0:2] + u[:, 2:4]
    return w[:, 0:1] + w[:, 1:2]


def _half_min_argmin(dist, iota_f):
    minv = jnp.min(dist, axis=1, keepdims=True)
    idx_f = jnp.min(jnp.where(dist == minv, iota_f, 1e9), axis=1)
    return minv[:, 0], idx_f.astype(jnp.int32)


def _dist_argmin_kernel(x_ref, cb_ref, idx_ref, loss_ref, s2_ref, iota_ref):
    i = pl.program_id(0)

    @pl.when(i == 0)
    def _precompute():
        c = cb_ref[...]
        s2_ref[...] = _row_sumsq(c).reshape(1, NUM_CODES)
        iota_ref[...] = lax.broadcasted_iota(
            jnp.int32, (1, NUM_CODES), 1).astype(jnp.float32)

    x = x_ref[...]
    s1 = _row_sumsq(x)
    m2 = lax.dot_general(-2.0 * x, cb_ref[...], (((1,), (1,)), ((), ())),
                         preferred_element_type=jnp.float32)
    dist = (s1 + s2_ref[...]) + m2
    half = NUM_CODES // 2
    iota_f = iota_ref[...]
    vL, iL = _half_min_argmin(dist[:, :half], iota_f[:, :half])
    vR, iR = _half_min_argmin(dist[:, half:], iota_f[:, half:])
    vLb = vL.astype(jnp.bfloat16).astype(jnp.float32)
    pick_l = vLb <= vR
    idx = jnp.where(pick_l, iL, iR)
    minv = jnp.where(pick_l, vL, vR)
    idx_ref[0, 0, :] = idx

    @pl.when(i == 0)
    def _init():
        loss_ref[...] = jnp.zeros((1, 1), jnp.float32)

    loss_ref[...] += jnp.sum(minv).reshape(1, 1)

    @pl.when(i == pl.num_programs(0) - 1)
    def _finalize():
        scale = (1.0 + COMMITMENT_COST) / (N_TOKENS * EMBED_DIM)
        loss_ref[...] = loss_ref[...] * scale


def _dist_argmin(inputs, codebook):
    n_tiles = N_TOKENS // TOKEN_TILE
    idx3, loss = pl.pallas_call(
        _dist_argmin_kernel,
        grid=(n_tiles,),
        in_specs=[
            pl.BlockSpec((TOKEN_TILE, EMBED_DIM), lambda i: (i, 0)),
            pl.BlockSpec((NUM_CODES, EMBED_DIM), lambda i: (0, 0)),
        ],
        out_specs=[
            pl.BlockSpec((1, 1, TOKEN_TILE), lambda i: (i, 0, 0)),
            pl.BlockSpec((1, 1), lambda i: (0, 0)),
        ],
        out_shape=[
            jax.ShapeDtypeStruct((n_tiles, 1, TOKEN_TILE), jnp.int32),
            jax.ShapeDtypeStruct((1, 1), jnp.float32),
        ],
        scratch_shapes=[
            pltpu.VMEM((1, NUM_CODES), jnp.float32),
            pltpu.VMEM((1, NUM_CODES), jnp.float32),
        ],
    )(inputs, codebook)
    return idx3.reshape(N_TOKENS), loss[0, 0]


GATHER_WIDTH = 128


def _make_sc_gather():
    info = plsc.get_sparse_core_info()
    nc, ns = info.num_cores, info.num_subcores
    nw = nc * ns
    b_per_w = N_TOKENS // nw
    chunk = 128
    n_chunks = b_per_w // chunk
    mesh = plsc.VectorSubcoreMesh(core_axis_name="c", subcore_axis_name="s")

    @functools.partial(
        pl.kernel,
        mesh=mesh,
        out_type=jax.ShapeDtypeStruct((N_TOKENS, GATHER_WIDTH), jnp.float32),
        scratch_types=[
            pltpu.VMEM((n_chunks, chunk), jnp.int32),
            pltpu.VMEM((chunk, GATHER_WIDTH), jnp.float32),
            pltpu.SemaphoreType.DMA,
        ],
    )
    def gather_kernel(table_hbm, idx_hbm, out_hbm, idx_v, rows_v, sem):
        wid = lax.axis_index("s") * nc + lax.axis_index("c")
        base = wid * b_per_w
        for j in range(n_chunks):
            pltpu.sync_copy(idx_hbm.at[pl.ds(base + j * chunk, chunk)],
                            idx_v.at[j])
            pltpu.async_copy(table_hbm.at[idx_v.at[j]], rows_v, sem).wait()
            pltpu.sync_copy(rows_v,
                            out_hbm.at[pl.ds(base + j * chunk, chunk)])

    return gather_kernel


def kernel(inputs, codebook):
    encoding_indices, loss = _dist_argmin(inputs, codebook)
    table = jnp.pad(codebook, ((0, 0), (0, GATHER_WIDTH - EMBED_DIM)))
    quantized = _make_sc_gather()(table, encoding_indices)[:, :EMBED_DIM]
    return (quantized, loss, encoding_indices)
